# Initial kernel scaffold; baseline (speedup 1.0000x reference)
import jax
import jax.numpy as jnp
from jax.experimental import pallas as pl


def kernel(input_ids, table):
    raise NotImplementedError("write your pallas kernel here")


# speedup vs baseline: 4.8081x; 4.8081x over previous
import functools

import jax
import jax.numpy as jnp
from jax import lax
from jax.experimental import pallas as pl
from jax.experimental.pallas import tpu as pltpu
from jax.experimental.pallas import tpu_sc as plsc

VOCAB = 1000000
EMBED_DIM = 32
NUM_IDS = 16384 * 200
NC, NS, L = 2, 16, 16
NW = NC * NS
IDS_PER_W = NUM_IDS // NW
CHUNK = 1024
ROWS_PER_GATHER = 128
GATHERS = CHUNK // ROWS_PER_GATHER
CHUNKS_PER_W = IDS_PER_W // CHUNK
IDS2_COLS = 128


def _embed_kernel(ids_hbm, table_hbm, out_hbm, idx_v, rows_v, sem):
    wid = lax.axis_index("s") * NC + lax.axis_index("c")
    id_row_base = wid * (IDS_PER_W // IDS2_COLS)
    out_row_base = wid * IDS_PER_W

    zeros16 = jnp.zeros((L,), jnp.float32)
    lane = lax.iota(jnp.int32, L)

    def chunk_body(g, _):
        pltpu.sync_copy(ids_hbm.at[pl.ds(id_row_base + g * GATHERS, GATHERS)],
                        idx_v)

        copies = []
        for j in range(GATHERS):
            copies.append(pltpu.async_copy(
                table_hbm.at[idx_v.at[j]],
                rows_v.at[pl.ds(j * ROWS_PER_GATHER, ROWS_PER_GATHER)],
                sem))
        for c in copies:
            c.wait()

        acc = jnp.full((L,), VOCAB, jnp.int32)
        for r in range(GATHERS):
            for c16 in range(IDS2_COLS // L):
                acc = jnp.minimum(acc, idx_v[r, pl.ds(c16 * L, L)])
        cnt = plsc.all_reduce_population_count(acc == 0)
        has_pad = cnt[0] > 0

        @pl.when(has_pad)
        def _fix():
            def fix_row(r, _):
                for c16 in range(IDS2_COLS // L):
                    vec = idx_v[r, pl.ds(c16 * L, L)]
                    mask = vec == 0
                    row_ids = (r * IDS2_COLS + c16 * L) + lane
                    for col in range(EMBED_DIM):
                        col_ids = jnp.full((L,), col, jnp.int32)
                        plsc.store_scatter(rows_v, [row_ids, col_ids],
                                           zeros16, mask=mask)
                return _
            lax.fori_loop(0, GATHERS, fix_row, 0)

        pltpu.sync_copy(rows_v,
                        out_hbm.at[pl.ds(out_row_base + g * CHUNK, CHUNK)])
        return _

    lax.fori_loop(0, CHUNKS_PER_W, chunk_body, 0)


@functools.partial(jax.jit, static_argnames=())
def kernel(input_ids, table):
    ids2 = input_ids.reshape(NUM_IDS // IDS2_COLS, IDS2_COLS)
    mesh = plsc.VectorSubcoreMesh(core_axis_name="c", subcore_axis_name="s",
                                  num_cores=NC, num_subcores=NS)
    out = pl.kernel(
        _embed_kernel,
        out_type=jax.ShapeDtypeStruct((NUM_IDS, EMBED_DIM), jnp.float32),
        mesh=mesh,
        scratch_types=[
            pltpu.VMEM((GATHERS, IDS2_COLS), jnp.int32),
            pltpu.VMEM((CHUNK, EMBED_DIM), jnp.float32),
            pltpu.SemaphoreType.DMA,
        ],
        compiler_params=pltpu.CompilerParams(needs_layout_passes=False,
                                             use_tc_tiling_on_sc=False),
    )(ids2, table)
    return out.reshape(input_ids.shape[0], input_ids.shape[1], EMBED_DIM)


# speedup vs baseline: 5.0358x; 1.0474x over previous
import functools

import jax
import jax.numpy as jnp
from jax import lax
from jax.experimental import pallas as pl
from jax.experimental.pallas import tpu as pltpu
from jax.experimental.pallas import tpu_sc as plsc

VOCAB = 1000000
EMBED_DIM = 32
SEQS = 16384
SEQ_LEN = 200
NC, NS, L = 2, 16, 16
NW = NC * NS
SEQS_PER_W = SEQS // NW
NSEQ = 8
CHUNKS_PER_W = SEQS_PER_W // NSEQ
GSPLIT = (128, 72)
NBUF = 2


def _embed_kernel(ids_hbm, table_hbm, out_hbm, idx_v, rows_v, gsem, osem):
    wid = lax.axis_index("s") * NC + lax.axis_index("c")
    seq_base = wid * SEQS_PER_W

    zeros16 = jnp.zeros((L,), jnp.float32)
    lane = lax.iota(jnp.int32, L)
    offs = [c * L for c in range(SEQ_LEN // L)] + [SEQ_LEN - L]

    def stage_idx(g, b):
        pltpu.sync_copy(ids_hbm.at[pl.ds(seq_base + g * NSEQ, NSEQ)],
                        idx_v.at[b])

    def gather_copies(b, make_only):
        mk = pltpu.make_async_copy if make_only else pltpu.async_copy
        for j in range(NSEQ):
            o = 0
            for glen in GSPLIT:
                mk(table_hbm.at[idx_v.at[b, j, pl.ds(o, glen)]],
                   rows_v.at[b, j, pl.ds(o, glen)],
                   gsem)
                o += glen

    def fire_gathers(b):
        gather_copies(b, make_only=False)

    def drain_gathers(b):
        for j in range(NSEQ):
            o = 0
            for glen in GSPLIT:
                pltpu.make_async_copy(
                    table_hbm.at[idx_v.at[b, j, pl.ds(o, glen)]],
                    rows_v.at[b, j, pl.ds(o, glen)],
                    gsem).wait()
                o += glen

    def out_slice(g):
        return out_hbm.at[pl.ds(seq_base + g * NSEQ, NSEQ)]

    def fire_out(g, b):
        pltpu.async_copy(rows_v.at[b], out_slice(g), osem)

    def wait_out(g, b):
        pltpu.make_async_copy(rows_v.at[b], out_slice(g), osem).wait()

    def scan_and_fix(b):
        acc = jnp.full((L,), VOCAB, jnp.int32)
        for j in range(NSEQ):
            for o in offs:
                acc = jnp.minimum(acc, idx_v[b, j, pl.ds(o, L)])
        cnt = plsc.all_reduce_population_count(acc == 0)
        has_pad = cnt[0] > 0

        @pl.when(has_pad)
        def _fix():
            bvec = jnp.full((L,), b, jnp.int32)

            def fix_row(j, _):
                jvec = jnp.full((L,), j, jnp.int32)
                for o in offs:
                    vec = idx_v[b, j, pl.ds(o, L)]
                    mask = vec == 0
                    row_ids = o + lane
                    for col in range(EMBED_DIM):
                        col_ids = jnp.full((L,), col, jnp.int32)
                        plsc.store_scatter(rows_v,
                                           [bvec, jvec, row_ids, col_ids],
                                           zeros16, mask=mask)
                return _
            lax.fori_loop(0, NSEQ, fix_row, 0)

    stage_idx(0, 0)
    fire_gathers(0)

    def chunk_body(g, _):
        b = g % NBUF
        nb = (g + 1) % NBUF

        drain_gathers(b)
        scan_and_fix(b)
        fire_out(g, b)

        @pl.when(g + 1 < CHUNKS_PER_W)
        def _next():
            @pl.when(g >= 1)
            def _w():
                wait_out(g - 1, nb)
            stage_idx(g + 1, nb)
            fire_gathers(nb)
        return _

    lax.fori_loop(0, CHUNKS_PER_W, chunk_body, 0)

    gl = CHUNKS_PER_W - 1
    wait_out(gl - 1, (gl - 1) % NBUF)
    wait_out(gl, gl % NBUF)


@functools.partial(jax.jit, static_argnames=())
def kernel(input_ids, table):
    mesh = plsc.VectorSubcoreMesh(core_axis_name="c", subcore_axis_name="s",
                                  num_cores=NC, num_subcores=NS)
    out = pl.kernel(
        _embed_kernel,
        out_type=jax.ShapeDtypeStruct((SEQS, SEQ_LEN, EMBED_DIM),
                                      jnp.float32),
        mesh=mesh,
        scratch_types=[
            pltpu.VMEM((NBUF, NSEQ, SEQ_LEN), jnp.int32),
            pltpu.VMEM((NBUF, NSEQ, SEQ_LEN, EMBED_DIM), jnp.float32),
            pltpu.SemaphoreType.DMA,
            pltpu.SemaphoreType.DMA,
        ],
        compiler_params=pltpu.CompilerParams(needs_layout_passes=False,
                                             use_tc_tiling_on_sc=False),
    )(input_ids, table)
    return out
